# Initial kernel scaffold; baseline (speedup 1.0000x reference)
import jax
import jax.numpy as jnp
from jax.experimental import pallas as pl


def kernel(vis, lang, state, W_f, b_f, lnf_g, lnf_b, W_g, W1, b1, W2, b2, W3, b3, ln_g, ln_b):
    raise NotImplementedError("write your pallas kernel here")


# speedup vs baseline: 2.7747x; 2.7747x over previous
import functools

import jax
import jax.numpy as jnp
from jax import lax
from jax.experimental import pallas as pl
from jax.experimental.pallas import tpu as pltpu
from jax.experimental.pallas import tpu_sc as plsc

B = 2048
D = 768
E = 16
TOPK = 1
BLK = 128
NB = 32
P = NB * BLK
BA = 256
NC, NS = 2, 16
NW = NC * NS


def _gelu(x):
    return 0.5 * x * (1.0 + lax.erf(x * 0.7071067811865476))


def _fusion_body(xc_ref, wf_ref, bf_ref, g_ref, bb_ref, wg_ref,
                 x_ref, ti_ref, ps_ref):
    h = jnp.dot(xc_ref[...], wf_ref[...], preferred_element_type=jnp.float32)
    h = h + bf_ref[...]
    m = jnp.mean(h, axis=-1, keepdims=True)
    v = jnp.mean((h - m) ** 2, axis=-1, keepdims=True)
    h = (h - m) / jnp.sqrt(v + 1e-5) * g_ref[...] + bb_ref[...]
    x = _gelu(h)
    x_ref[...] = x
    lg = jnp.dot(x, wg_ref[...], preferred_element_type=jnp.float32)
    mx = jnp.max(lg, axis=-1, keepdims=True)
    ids = lax.broadcasted_iota(jnp.int32, lg.shape, 1)
    ti_ref[0, 0, :] = jnp.min(jnp.where(lg >= mx, ids, E), axis=-1)
    p = jnp.exp(lg - mx)
    p = p / jnp.sum(p, axis=-1, keepdims=True)
    ps_ref[0, 0, :] = jnp.sum(p, axis=0)


def _fusion(xcat, W_f, b_f2, lnf_g2, lnf_b2, W_g):
    nblk = B // BA
    d_in = xcat.shape[1]
    return pl.pallas_call(
        _fusion_body,
        grid=(nblk,),
        in_specs=[
            pl.BlockSpec((BA, d_in), lambda i: (i, 0)),
            pl.BlockSpec((d_in, D), lambda i: (0, 0)),
            pl.BlockSpec((1, D), lambda i: (0, 0)),
            pl.BlockSpec((1, D), lambda i: (0, 0)),
            pl.BlockSpec((1, D), lambda i: (0, 0)),
            pl.BlockSpec((D, E), lambda i: (0, 0)),
        ],
        out_specs=[
            pl.BlockSpec((BA, D), lambda i: (i, 0)),
            pl.BlockSpec((1, 1, BA), lambda i: (i, 0, 0)),
            pl.BlockSpec((1, 1, E), lambda i: (i, 0, 0)),
        ],
        out_shape=[
            jax.ShapeDtypeStruct((B, D), jnp.float32),
            jax.ShapeDtypeStruct((nblk, 1, BA), jnp.int32),
            jax.ShapeDtypeStruct((nblk, 1, E), jnp.float32),
        ],
    )(xcat, W_f, b_f2, lnf_g2, lnf_b2, W_g)


def _expert_body(eid_ref, size_ref, xs_ref, w1_ref, b1_ref, w2_ref, b2_ref,
                 w3_ref, b3_ref, g_ref, bb_ref, out_ref):
    i = pl.program_id(0)

    @pl.when(size_ref[i] > 0)
    def _():
        x = xs_ref[...]
        h = jnp.dot(x, w1_ref[0], preferred_element_type=jnp.float32)
        h = _gelu(h + b1_ref[0])
        h = jnp.dot(h, w2_ref[0], preferred_element_type=jnp.float32)
        h = _gelu(h + b2_ref[0])
        h = jnp.dot(h, w3_ref[0], preferred_element_type=jnp.float32)
        y = x + h + b3_ref[0]
        m = jnp.mean(y, axis=-1, keepdims=True)
        v = jnp.mean((y - m) ** 2, axis=-1, keepdims=True)
        out_ref[...] = (y - m) / jnp.sqrt(v + 1e-5) * g_ref[0] + bb_ref[0]


def _experts(eids, sizes, xs, W1, b1r, W2, b2r, W3, b3r, ln_gr, ln_br):
    grid_spec = pltpu.PrefetchScalarGridSpec(
        num_scalar_prefetch=2,
        grid=(NB,),
        in_specs=[
            pl.BlockSpec((BLK, D), lambda i, e, s: (i, 0)),
            pl.BlockSpec((1, D, 2 * D), lambda i, e, s: (e[i], 0, 0)),
            pl.BlockSpec((1, 1, 2 * D), lambda i, e, s: (e[i], 0, 0)),
            pl.BlockSpec((1, 2 * D, D), lambda i, e, s: (e[i], 0, 0)),
            pl.BlockSpec((1, 1, D), lambda i, e, s: (e[i], 0, 0)),
            pl.BlockSpec((1, D, D), lambda i, e, s: (e[i], 0, 0)),
            pl.BlockSpec((1, 1, D), lambda i, e, s: (e[i], 0, 0)),
            pl.BlockSpec((1, 1, D), lambda i, e, s: (e[i], 0, 0)),
            pl.BlockSpec((1, 1, D), lambda i, e, s: (e[i], 0, 0)),
        ],
        out_specs=pl.BlockSpec((BLK, D), lambda i, e, s: (i, 0)),
    )
    return pl.pallas_call(
        _expert_body,
        grid_spec=grid_spec,
        out_shape=jax.ShapeDtypeStruct((P, D), jnp.float32),
    )(eids, sizes, xs, W1, b1r, W2, b2r, W3, b3r, ln_gr, ln_br)


def _sc_gather(x, idx):
    b_per_w = P // NW
    mesh = plsc.VectorSubcoreMesh(core_axis_name="c", subcore_axis_name="s")

    def body(x_hbm, idx_hbm, out_hbm, idx_v, rows_v, sem):
        wid = lax.axis_index("s") * NC + lax.axis_index("c")
        base = wid * b_per_w
        pltpu.sync_copy(idx_hbm.at[pl.ds(base, b_per_w)], idx_v)
        pltpu.async_copy(x_hbm.at[idx_v], rows_v, sem).wait()
        pltpu.sync_copy(rows_v, out_hbm.at[pl.ds(base, b_per_w)])

    return pl.kernel(
        body,
        out_type=jax.ShapeDtypeStruct((P, D), jnp.float32),
        mesh=mesh,
        scratch_types=[
            pltpu.VMEM((b_per_w,), jnp.int32),
            pltpu.VMEM((b_per_w, D), jnp.float32),
            pltpu.SemaphoreType.DMA,
        ],
    )(x, idx)


def _sc_scatter(ys, idx):
    b_per_w = P // NW
    mesh = plsc.VectorSubcoreMesh(core_axis_name="c", subcore_axis_name="s")

    def body(ys_hbm, idx_hbm, out_hbm, idx_v, rows_v, sem):
        wid = lax.axis_index("s") * NC + lax.axis_index("c")
        base = wid * b_per_w
        pltpu.sync_copy(idx_hbm.at[pl.ds(base, b_per_w)], idx_v)
        pltpu.sync_copy(ys_hbm.at[pl.ds(base, b_per_w)], rows_v)
        pltpu.async_copy(rows_v, out_hbm.at[idx_v], sem).wait()

    return pl.kernel(
        body,
        out_type=jax.ShapeDtypeStruct((B + 8, D), jnp.float32),
        mesh=mesh,
        scratch_types=[
            pltpu.VMEM((b_per_w,), jnp.int32),
            pltpu.VMEM((b_per_w, D), jnp.float32),
            pltpu.SemaphoreType.DMA,
        ],
    )(ys, idx)


def kernel(vis, lang, state, W_f, b_f, lnf_g, lnf_b, W_g, W1, b1, W2, b2,
           W3, b3, ln_g, ln_b):
    xcat = jnp.concatenate([vis, lang, state], axis=-1)
    x, ti3, ps3 = _fusion(xcat, W_f, b_f.reshape(1, D), lnf_g.reshape(1, D),
                          lnf_b.reshape(1, D), W_g)
    top_idx = ti3.reshape(B)

    sorted_e, sort_ids = lax.sort_key_val(top_idx, jnp.arange(B, dtype=jnp.int32))
    starts = jnp.searchsorted(sorted_e, jnp.arange(E + 1, dtype=jnp.int32),
                              side="left").astype(jnp.int32)
    counts = starts[1:] - starts[:-1]
    nb = (counts + BLK - 1) // BLK
    cum_nb = jnp.cumsum(nb).astype(jnp.int32)
    bids = jnp.arange(NB, dtype=jnp.int32)
    blk_e = jnp.searchsorted(cum_nb, bids, side="right").astype(jnp.int32)
    valid_blk = bids < cum_nb[-1]
    blk_e_c = jnp.minimum(blk_e, E - 1)
    j = bids - (cum_nb[blk_e_c] - nb[blk_e_c])
    blk_start = starts[blk_e_c] + j * BLK
    blk_size = jnp.clip(counts[blk_e_c] - j * BLK, 0, BLK)
    blk_size = jnp.where(valid_blk, blk_size, 0).astype(jnp.int32)
    eids = jnp.where(valid_blk, blk_e_c, sorted_e[B - 1]).astype(jnp.int32)

    p = jnp.arange(P, dtype=jnp.int32)
    bi = p // BLK
    lane = p % BLK
    pos = jnp.clip(blk_start[bi] + lane, 0, B - 1)
    row = sort_ids[pos]
    valid = lane < blk_size[bi]
    src = jnp.where(valid, row, 0)
    dst = jnp.where(valid, row, B)

    xs = _sc_gather(x, src)
    ys = _experts(eids, blk_size, xs, W1, b1.reshape(E, 1, 2 * D), W2,
                  b2.reshape(E, 1, D), W3, b3.reshape(E, 1, D),
                  ln_g.reshape(E, 1, D), ln_b.reshape(E, 1, D))
    out = _sc_scatter(ys, dst)[:B]

    mean_probs = jnp.sum(ps3.reshape(-1, E), axis=0) / B
    lb_loss = E * jnp.sum(counts.astype(jnp.float32) / (B * TOPK) * mean_probs)
    return (out, lb_loss)


# speedup vs baseline: 4.5069x; 1.6243x over previous
import functools

import jax
import jax.numpy as jnp
from jax import lax
from jax.experimental import pallas as pl
from jax.experimental.pallas import tpu as pltpu
from jax.experimental.pallas import tpu_sc as plsc

B = 2048
D = 768
E = 16
TOPK = 1
BLK = 128
NB = 32
P = NB * BLK
BA = 256
NC, NS = 2, 16
NW = NC * NS


def _gelu(x):
    return 0.5 * x * (1.0 + lax.erf(x * 0.7071067811865476))


def _fusion_body(xc_ref, wf_ref, bf_ref, g_ref, bb_ref, wg_ref,
                 x_ref, ti_ref, ps_ref):
    h = jnp.dot(xc_ref[...], wf_ref[...], preferred_element_type=jnp.float32)
    h = h + bf_ref[...]
    m = jnp.mean(h, axis=-1, keepdims=True)
    v = jnp.mean((h - m) ** 2, axis=-1, keepdims=True)
    h = (h - m) / jnp.sqrt(v + 1e-5) * g_ref[...] + bb_ref[...]
    x = _gelu(h)
    x_ref[...] = x
    lg = jnp.dot(x, wg_ref[...], preferred_element_type=jnp.float32)
    mx = jnp.max(lg, axis=-1, keepdims=True)
    ids = lax.broadcasted_iota(jnp.int32, lg.shape, 1)
    ti_ref[0, 0, :] = jnp.min(jnp.where(lg >= mx, ids, E), axis=-1)
    p = jnp.exp(lg - mx)
    p = p / jnp.sum(p, axis=-1, keepdims=True)
    ps_ref[0, 0, :] = jnp.sum(p, axis=0)


def _fusion(xcat, W_f, b_f2, lnf_g2, lnf_b2, W_g):
    nblk = B // BA
    d_in = xcat.shape[1]
    return pl.pallas_call(
        _fusion_body,
        grid=(nblk,),
        in_specs=[
            pl.BlockSpec((BA, d_in), lambda i: (i, 0)),
            pl.BlockSpec((d_in, D), lambda i: (0, 0)),
            pl.BlockSpec((1, D), lambda i: (0, 0)),
            pl.BlockSpec((1, D), lambda i: (0, 0)),
            pl.BlockSpec((1, D), lambda i: (0, 0)),
            pl.BlockSpec((D, E), lambda i: (0, 0)),
        ],
        out_specs=[
            pl.BlockSpec((BA, D), lambda i: (i, 0)),
            pl.BlockSpec((1, 1, BA), lambda i: (i, 0, 0)),
            pl.BlockSpec((1, 1, E), lambda i: (i, 0, 0)),
        ],
        out_shape=[
            jax.ShapeDtypeStruct((B, D), jnp.float32),
            jax.ShapeDtypeStruct((nblk, 1, BA), jnp.int32),
            jax.ShapeDtypeStruct((nblk, 1, E), jnp.float32),
        ],
    )(xcat, W_f, b_f2, lnf_g2, lnf_b2, W_g)


def _expert_body(eid_ref, size_ref, xs_ref, w1_ref, b1_ref, w2_ref, b2_ref,
                 w3_ref, b3_ref, g_ref, bb_ref, out_ref):
    i = pl.program_id(0)

    @pl.when(size_ref[i] > 0)
    def _():
        x = xs_ref[...]
        h = jnp.dot(x.astype(jnp.bfloat16), w1_ref[0].astype(jnp.bfloat16),
                    preferred_element_type=jnp.float32)
        h = _gelu(h + b1_ref[0])
        h = jnp.dot(h.astype(jnp.bfloat16), w2_ref[0].astype(jnp.bfloat16),
                    preferred_element_type=jnp.float32)
        h = _gelu(h + b2_ref[0])
        h = jnp.dot(h.astype(jnp.bfloat16), w3_ref[0].astype(jnp.bfloat16),
                    preferred_element_type=jnp.float32)
        y = x + h + b3_ref[0]
        m = jnp.mean(y, axis=-1, keepdims=True)
        v = jnp.mean((y - m) ** 2, axis=-1, keepdims=True)
        out_ref[...] = (y - m) / jnp.sqrt(v + 1e-5) * g_ref[0] + bb_ref[0]


def _experts(eids, sizes, xs, W1, b1r, W2, b2r, W3, b3r, ln_gr, ln_br):
    grid_spec = pltpu.PrefetchScalarGridSpec(
        num_scalar_prefetch=2,
        grid=(NB,),
        in_specs=[
            pl.BlockSpec((BLK, D), lambda i, e, s: (i, 0)),
            pl.BlockSpec((1, D, 2 * D), lambda i, e, s: (e[i], 0, 0)),
            pl.BlockSpec((1, 1, 2 * D), lambda i, e, s: (e[i], 0, 0)),
            pl.BlockSpec((1, 2 * D, D), lambda i, e, s: (e[i], 0, 0)),
            pl.BlockSpec((1, 1, D), lambda i, e, s: (e[i], 0, 0)),
            pl.BlockSpec((1, D, D), lambda i, e, s: (e[i], 0, 0)),
            pl.BlockSpec((1, 1, D), lambda i, e, s: (e[i], 0, 0)),
            pl.BlockSpec((1, 1, D), lambda i, e, s: (e[i], 0, 0)),
            pl.BlockSpec((1, 1, D), lambda i, e, s: (e[i], 0, 0)),
        ],
        out_specs=pl.BlockSpec((BLK, D), lambda i, e, s: (i, 0)),
    )
    return pl.pallas_call(
        _expert_body,
        grid_spec=grid_spec,
        out_shape=jax.ShapeDtypeStruct((P, D), jnp.float32),
    )(eids, sizes, xs, W1, b1r, W2, b2r, W3, b3r, ln_gr, ln_br)


def _sc_gather(x, idx):
    b_per_w = P // NW
    mesh = plsc.VectorSubcoreMesh(core_axis_name="c", subcore_axis_name="s")

    def body(x_hbm, idx_hbm, out_hbm, idx_v, rows_v, sem):
        wid = lax.axis_index("s") * NC + lax.axis_index("c")
        base = wid * b_per_w
        pltpu.sync_copy(idx_hbm.at[pl.ds(base, b_per_w)], idx_v)
        pltpu.async_copy(x_hbm.at[idx_v], rows_v, sem).wait()
        pltpu.sync_copy(rows_v, out_hbm.at[pl.ds(base, b_per_w)])

    return pl.kernel(
        body,
        out_type=jax.ShapeDtypeStruct((P, D), jnp.float32),
        mesh=mesh,
        scratch_types=[
            pltpu.VMEM((b_per_w,), jnp.int32),
            pltpu.VMEM((b_per_w, D), jnp.float32),
            pltpu.SemaphoreType.DMA,
        ],
    )(x, idx)


def _sc_scatter(ys, idx):
    b_per_w = P // NW
    mesh = plsc.VectorSubcoreMesh(core_axis_name="c", subcore_axis_name="s")

    def body(ys_hbm, idx_hbm, out_hbm, idx_v, rows_v, sem):
        wid = lax.axis_index("s") * NC + lax.axis_index("c")
        base = wid * b_per_w
        pltpu.sync_copy(idx_hbm.at[pl.ds(base, b_per_w)], idx_v)
        pltpu.sync_copy(ys_hbm.at[pl.ds(base, b_per_w)], rows_v)
        pltpu.async_copy(rows_v, out_hbm.at[idx_v], sem).wait()

    return pl.kernel(
        body,
        out_type=jax.ShapeDtypeStruct((B + P, D), jnp.float32),
        mesh=mesh,
        scratch_types=[
            pltpu.VMEM((b_per_w,), jnp.int32),
            pltpu.VMEM((b_per_w, D), jnp.float32),
            pltpu.SemaphoreType.DMA,
        ],
    )(ys, idx)


def kernel(vis, lang, state, W_f, b_f, lnf_g, lnf_b, W_g, W1, b1, W2, b2,
           W3, b3, ln_g, ln_b):
    xcat = jnp.concatenate([vis, lang, state], axis=-1)
    x, ti3, ps3 = _fusion(xcat, W_f, b_f.reshape(1, D), lnf_g.reshape(1, D),
                          lnf_b.reshape(1, D), W_g)
    top_idx = ti3.reshape(B)

    sorted_e, sort_ids = lax.sort_key_val(top_idx, jnp.arange(B, dtype=jnp.int32))
    starts = jnp.searchsorted(sorted_e, jnp.arange(E + 1, dtype=jnp.int32),
                              side="left").astype(jnp.int32)
    counts = starts[1:] - starts[:-1]
    nb = (counts + BLK - 1) // BLK
    cum_nb = jnp.cumsum(nb).astype(jnp.int32)
    bids = jnp.arange(NB, dtype=jnp.int32)
    blk_e = jnp.searchsorted(cum_nb, bids, side="right").astype(jnp.int32)
    valid_blk = bids < cum_nb[-1]
    blk_e_c = jnp.minimum(blk_e, E - 1)
    j = bids - (cum_nb[blk_e_c] - nb[blk_e_c])
    blk_start = starts[blk_e_c] + j * BLK
    blk_size = jnp.clip(counts[blk_e_c] - j * BLK, 0, BLK)
    blk_size = jnp.where(valid_blk, blk_size, 0).astype(jnp.int32)
    eids = jnp.where(valid_blk, blk_e_c, sorted_e[B - 1]).astype(jnp.int32)

    p = jnp.arange(P, dtype=jnp.int32)
    bi = p // BLK
    lane = p % BLK
    pos = jnp.clip(blk_start[bi] + lane, 0, B - 1)
    row = sort_ids[pos]
    valid = lane < blk_size[bi]
    src = jnp.where(valid, row, p % B)
    dst = jnp.where(valid, row, B + p)

    xs = _sc_gather(x, src)
    ys = _experts(eids, blk_size, xs, W1, b1.reshape(E, 1, 2 * D), W2,
                  b2.reshape(E, 1, D), W3, b3.reshape(E, 1, D),
                  ln_g.reshape(E, 1, D), ln_b.reshape(E, 1, D))
    out = _sc_scatter(ys, dst)[:B]

    mean_probs = jnp.sum(ps3.reshape(-1, E), axis=0) / B
    lb_loss = E * jnp.sum(counts.astype(jnp.float32) / (B * TOPK) * mean_probs)
    return (out, lb_loss)
